# baseline (speedup 1.0000x reference)
import jax
import jax.numpy as jnp
import numpy as np
from jax.experimental import pallas as pl
from jax.experimental.pallas import tpu as pltpu

_BN_INV = float(1.0 / np.sqrt(1.0 + 1e-5))
_F32 = jnp.float32


def _pool(x):
    A, B, C = x.shape
    x = x.reshape(A // 2, 2, B, C)
    x = jnp.maximum(x[:, 0], x[:, 1])
    x = jnp.swapaxes(x, 0, 1)
    x = x.reshape(B // 2, 2, A // 2, C)
    return jnp.maximum(x[:, 0], x[:, 1])


def _conv_block(h, w_ref, st_ref, H, Cin, Cout, swapped):
    hp = jnp.pad(h, ((1, 1), (1, 1), (0, 0)))
    acc = None
    for dy in range(3):
        for dx in range(3):
            a, b = (dx, dy) if swapped else (dy, dx)
            sl = hp[a:a + H, b:b + H, :].reshape(H * H, Cin)
            p = jnp.dot(sl, w_ref[dy * 3 + dx], preferred_element_type=_F32)
            acc = p if acc is None else acc + p
    y = jnp.maximum(acc * st_ref[0:1, :] + st_ref[1:2, :], 0.0)
    return _pool(y.reshape(H, H, Cout))


def _enc_body(x9_ref, w9_ref, st1_ref, w2_ref, st2_ref, w3_ref, st3_ref,
              w4_ref, st4_ref, attw_ref, attb_ref, wA_ref, wB_ref, bi0_ref,
              attp_ref, pA_ref, pB_ref):
    x9 = x9_ref[0]
    c = jnp.dot(x9, w9_ref[...], preferred_element_type=_F32)
    c = jnp.maximum(c * st1_ref[0:1, :] + st1_ref[1:2, :], 0.0)
    h = _pool(c.reshape(112, 112, 32))
    h = _conv_block(h, w2_ref, st2_ref, 56, 32, 64, True)
    h = _conv_block(h, w3_ref, st3_ref, 28, 64, 128, False)
    h = _conv_block(h, w4_ref, st4_ref, 14, 128, 256, True)
    feat = h.reshape(49, 256)
    attp_ref[0] = jnp.dot(feat, attw_ref[...],
                          preferred_element_type=_F32) + attb_ref[...]
    pA_ref[0] = jnp.dot(feat, wA_ref[...], preferred_element_type=_F32)
    pB_ref[0] = jnp.dot(feat, wB_ref[...],
                        preferred_element_type=_F32) + bi0_ref[...]


def _att_body(wq_ref, wh_ref, hfAa_ref, qfB_ref, out_ref):
    wq = wq_ref[0]
    wh = wh_ref[0]
    prod = wq[:, None, :] * wh[None, :, :]
    scores = jnp.sum(jnp.tanh(prod), axis=-1)
    e = jnp.exp(scores)
    nd = jnp.dot(e, hfAa_ref[0], preferred_element_type=_F32)
    out_ref[:, 0, 0] = nd[:, :384] / nd[:, 384:385] + qfB_ref[0]


def _gru_body(gi_ref, wh0_ref, bh0_ref, wi1_ref, bi1_ref, wh1_ref, bh1_ref,
              tw_ref, out_ref, h0_ref, h1_ref):
    t = pl.program_id(1)

    @pl.when(t == 0)
    def _():
        h0_ref[...] = jnp.zeros_like(h0_ref)
        h1_ref[...] = jnp.zeros_like(h1_ref)

    gi = gi_ref[0]
    h0 = h0_ref[...]
    gh = jnp.dot(h0, wh0_ref[...], preferred_element_type=_F32) + bh0_ref[...]
    r = jax.nn.sigmoid(gi[:, 0:128] + gh[:, 0:128])
    z = jax.nn.sigmoid(gi[:, 128:256] + gh[:, 128:256])
    n = jnp.tanh(gi[:, 256:384] + r * gh[:, 256:384])
    h0 = (1.0 - z) * n + z * h0
    h0_ref[...] = h0

    gi1 = jnp.dot(h0, wi1_ref[...], preferred_element_type=_F32) + bi1_ref[...]
    h1 = h1_ref[...]
    gh1 = jnp.dot(h1, wh1_ref[...], preferred_element_type=_F32) + bh1_ref[...]
    r1 = jax.nn.sigmoid(gi1[:, 0:128] + gh1[:, 0:128])
    z1 = jax.nn.sigmoid(gi1[:, 128:256] + gh1[:, 128:256])
    n1 = jnp.tanh(gi1[:, 256:384] + r1 * gh1[:, 256:384])
    h1 = (1.0 - z1) * n1 + z1 * h1
    h1_ref[...] = h1

    @pl.when(t == 48)
    def _():
        out_ref[0] = jax.lax.dot_general(
            tw_ref[...], h1, (((1,), (1,)), ((), ())),
            preferred_element_type=_F32)


def _head_body(L_ref, out_ref):
    L = L_ref[...]
    u = jax.lax.broadcasted_iota(jnp.int32, (25, 5), 0)
    c = jax.lax.broadcasted_iota(jnp.int32, (25, 5), 1)
    ksum = jnp.where(u // 5 == c, 1.0, 0.0).astype(_F32)
    S = jnp.dot(L, ksum, preferred_element_type=_F32)
    m = jnp.max(S, axis=1, keepdims=True)
    e = jnp.exp(S - m)
    out_ref[...] = (S - m) - jnp.log(jnp.sum(e, axis=1, keepdims=True))


def kernel(support, query, conv_w1, conv_b1, conv_w2, conv_b2, conv_w3,
           conv_b3, conv_w4, conv_b4, bn_g1, bn_be1, bn_g2, bn_be2, bn_g3,
           bn_be3, bn_g4, bn_be4, att_w, att_b, gru_wi0, gru_wh0, gru_bi0,
           gru_bh0, gru_wi1, gru_wh1, gru_bi1, gru_bh1, trans_w, trans_b):
    del trans_b

    imgs = jnp.concatenate(
        [support.reshape(25, 224, 224), query.reshape(15, 224, 224)], axis=0)
    xpad = jnp.pad(imgs, ((0, 0), (1, 1), (1, 1)))
    taps = [xpad[:, dy:dy + 223:2, dx:dx + 223:2]
            for dy in range(3) for dx in range(3)]
    x9 = jnp.stack(taps, axis=-1).reshape(40, 12544, 9)

    def stp(g, be, b):
        s = g * _BN_INV
        return jnp.stack([s, b * s + be])

    w9 = conv_w1.reshape(32, 9).T
    st1 = stp(bn_g1, bn_be1, conv_b1)
    w2r = conv_w2.transpose(2, 3, 1, 0).reshape(9, 32, 64)
    st2 = stp(bn_g2, bn_be2, conv_b2)
    w3r = conv_w3.transpose(2, 3, 1, 0).reshape(9, 64, 128)
    st3 = stp(bn_g3, bn_be3, conv_b3)
    w4r = conv_w4.transpose(2, 3, 1, 0).reshape(9, 128, 256)
    st4 = stp(bn_g4, bn_be4, conv_b4)
    attwT = att_w.T
    attb2 = att_b[None, :]
    wA = gru_wi0[:, :256].T
    wB = gru_wi0[:, 256:].T
    bi0 = gru_bi0[None, :]

    full = lambda shape: pl.BlockSpec(shape, lambda c, i: (0,) * len(shape))
    img = lambda nd: (lambda c, i: (c * 20 + i,) + (0,) * (nd - 1))
    attp, pA, pB = pl.pallas_call(
        _enc_body,
        grid=(2, 20),
        in_specs=[
            pl.BlockSpec((1, 12544, 9), img(3)),
            full((9, 32)), full((2, 32)),
            full((9, 32, 64)), full((2, 64)),
            full((9, 64, 128)), full((2, 128)),
            full((9, 128, 256)), full((2, 256)),
            full((256, 64)), full((1, 64)),
            full((256, 384)), full((256, 384)), full((1, 384)),
        ],
        out_specs=[
            pl.BlockSpec((1, 49, 64), img(3)),
            pl.BlockSpec((1, 49, 384), img(3)),
            pl.BlockSpec((1, 49, 384), img(3)),
        ],
        out_shape=[
            jax.ShapeDtypeStruct((40, 49, 64), _F32),
            jax.ShapeDtypeStruct((40, 49, 384), _F32),
            jax.ShapeDtypeStruct((40, 49, 384), _F32),
        ],
        compiler_params=pltpu.CompilerParams(
            dimension_semantics=("parallel", "arbitrary")),
        name="encoder",
    )(x9, w9, st1, w2r, st2, w3r, st3, w4r, st4, attwT, attb2, wA, wB, bi0)

    atth, attq = attp[:25], attp[25:]
    qfB = pB[25:]
    hfAa = jnp.concatenate(
        [pA[:25], jnp.ones((25, 49, 1), _F32)], axis=-1)

    pidx = lambda c, j: jnp.minimum(c * 188 + j, 374)
    gi = pl.pallas_call(
        _att_body,
        grid=(2, 188),
        in_specs=[
            pl.BlockSpec((1, 49, 64), lambda c, j: (pidx(c, j) // 25, 0, 0)),
            pl.BlockSpec((1, 49, 64), lambda c, j: (pidx(c, j) % 25, 0, 0)),
            pl.BlockSpec((1, 49, 385), lambda c, j: (pidx(c, j) % 25, 0, 0)),
            pl.BlockSpec((1, 49, 384), lambda c, j: (pidx(c, j) // 25, 0, 0)),
        ],
        out_specs=pl.BlockSpec((49, 1, 1, 384),
                               lambda c, j: (0, pidx(c, j), 0, 0)),
        out_shape=jax.ShapeDtypeStruct((49, 384, 1, 384), _F32),
        compiler_params=pltpu.CompilerParams(
            dimension_semantics=("parallel", "arbitrary")),
        name="attention",
    )(attq, atth, hfAa, qfB)

    gi_t = gi.reshape(49, 384, 384)

    wh0T = gru_wh0.T
    wi1T = gru_wi1.T
    wh1T = gru_wh1.T
    bh0 = gru_bh0[None, :]
    bi1 = gru_bi1[None, :]
    bh1 = gru_bh1[None, :]
    tw = trans_w

    wspec = lambda shape: pl.BlockSpec(shape, lambda c, t: (0,) * len(shape))
    h_out = pl.pallas_call(
        _gru_body,
        grid=(2, 49),
        in_specs=[
            pl.BlockSpec((1, 192, 384), lambda c, t: (t, c, 0)),
            wspec((128, 384)), wspec((1, 384)),
            wspec((128, 384)), wspec((1, 384)),
            wspec((128, 384)), wspec((1, 384)),
            wspec((1, 128)),
        ],
        out_specs=pl.BlockSpec((1, 1, 192), lambda c, t: (c, 0, 0)),
        out_shape=jax.ShapeDtypeStruct((2, 1, 192), _F32),
        scratch_shapes=[pltpu.VMEM((192, 128), _F32),
                        pltpu.VMEM((192, 128), _F32)],
        compiler_params=pltpu.CompilerParams(
            dimension_semantics=("parallel", "arbitrary")),
        name="gru",
    )(gi_t, wh0T, bh0, wi1T, bi1, wh1T, bh1, tw)

    Lq = h_out.reshape(384)[:375].reshape(15, 25)
    return pl.pallas_call(
        _head_body,
        out_shape=jax.ShapeDtypeStruct((15, 5), _F32),
        name="head",
    )(Lq)


# speedup vs baseline: 1.3628x; 1.3628x over previous
import jax
import jax.numpy as jnp
import numpy as np
from jax.experimental import pallas as pl
from jax.experimental.pallas import tpu as pltpu

_BN_INV = float(1.0 / np.sqrt(1.0 + 1e-5))
_F32 = jnp.float32


def _pool(x):
    A, B, C = x.shape
    x = x.reshape(A // 2, 2, B, C)
    x = jnp.maximum(x[:, 0], x[:, 1])
    x = jnp.swapaxes(x, 0, 1)
    x = x.reshape(B // 2, 2, A // 2, C)
    return jnp.maximum(x[:, 0], x[:, 1])


def _conv_block(h, w_ref, st_ref, H, Cin, Cout, swapped):
    hp = jnp.pad(h, ((1, 1), (1, 1), (0, 0)))
    acc = None
    for dy in range(3):
        for dx in range(3):
            a, b = (dx, dy) if swapped else (dy, dx)
            sl = hp[a:a + H, b:b + H, :].reshape(H * H, Cin)
            p = jnp.dot(sl, w_ref[dy * 3 + dx], preferred_element_type=_F32)
            acc = p if acc is None else acc + p
    y = jnp.maximum(acc * st_ref[0:1, :] + st_ref[1:2, :], 0.0)
    return _pool(y.reshape(H, H, Cout))


def _enc_body(x9_ref, w9_ref, st1_ref, w2_ref, st2_ref, w3_ref, st3_ref,
              w4_ref, st4_ref, attw_ref, attb_ref, wA_ref, wB_ref, bi0_ref,
              attp_ref, pA_ref, pB_ref):
    x9 = x9_ref[0]
    c = jnp.dot(x9, w9_ref[...], preferred_element_type=_F32)
    c = jnp.maximum(c * st1_ref[0:1, :] + st1_ref[1:2, :], 0.0)
    h = _pool(c.reshape(112, 112, 32))
    h = _conv_block(h, w2_ref, st2_ref, 56, 32, 64, True)
    h = _conv_block(h, w3_ref, st3_ref, 28, 64, 128, False)
    h = _conv_block(h, w4_ref, st4_ref, 14, 128, 256, True)
    feat = h.reshape(49, 256)
    attp_ref[0] = jnp.dot(feat, attw_ref[...],
                          preferred_element_type=_F32) + attb_ref[...]
    pA_ref[0] = jnp.dot(feat, wA_ref[...], preferred_element_type=_F32)
    pB_ref[0] = jnp.dot(feat, wB_ref[...],
                        preferred_element_type=_F32) + bi0_ref[...]


def _att_body(wq_ref, wh_ref, hfAa_ref, qfB_ref, out_ref):
    wq = wq_ref[0]
    wh = wh_ref[0]
    prod = wq[:, None, :] * wh[None, :, :]
    scores = jnp.sum(jnp.tanh(prod), axis=-1)
    e = jnp.exp(scores)
    nd = jnp.dot(e, hfAa_ref[0], preferred_element_type=_F32)
    out_ref[:, 0, 0] = nd[:, :384] / nd[:, 384:385] + qfB_ref[0]


def _gru_body(gi_ref, wh0_ref, bh0_ref, wi1_ref, bi1_ref, wh1_ref, bh1_ref,
              tw_ref, out_ref, h0_ref, h1_ref):
    t = pl.program_id(1)

    @pl.when(t == 0)
    def _():
        h0_ref[...] = jnp.zeros_like(h0_ref)
        h1_ref[...] = jnp.zeros_like(h1_ref)

    gi = gi_ref[0]
    h0 = h0_ref[...]
    gh = jnp.dot(h0, wh0_ref[...], preferred_element_type=_F32) + bh0_ref[...]
    r = jax.nn.sigmoid(gi[:, 0:128] + gh[:, 0:128])
    z = jax.nn.sigmoid(gi[:, 128:256] + gh[:, 128:256])
    n = jnp.tanh(gi[:, 256:384] + r * gh[:, 256:384])
    h0 = (1.0 - z) * n + z * h0
    h0_ref[...] = h0

    gi1 = jnp.dot(h0, wi1_ref[...], preferred_element_type=_F32) + bi1_ref[...]
    h1 = h1_ref[...]
    gh1 = jnp.dot(h1, wh1_ref[...], preferred_element_type=_F32) + bh1_ref[...]
    r1 = jax.nn.sigmoid(gi1[:, 0:128] + gh1[:, 0:128])
    z1 = jax.nn.sigmoid(gi1[:, 128:256] + gh1[:, 128:256])
    n1 = jnp.tanh(gi1[:, 256:384] + r1 * gh1[:, 256:384])
    h1 = (1.0 - z1) * n1 + z1 * h1
    h1_ref[...] = h1

    @pl.when(t == 48)
    def _():
        out_ref[0] = jax.lax.dot_general(
            tw_ref[...], h1, (((1,), (1,)), ((), ())),
            preferred_element_type=_F32)


def _head_body(L_ref, out_ref):
    L = L_ref[...]
    u = jax.lax.broadcasted_iota(jnp.int32, (25, 5), 0)
    c = jax.lax.broadcasted_iota(jnp.int32, (25, 5), 1)
    ksum = jnp.where(u // 5 == c, 1.0, 0.0).astype(_F32)
    S = jnp.dot(L, ksum, preferred_element_type=_F32)
    m = jnp.max(S, axis=1, keepdims=True)
    e = jnp.exp(S - m)
    out_ref[...] = (S - m) - jnp.log(jnp.sum(e, axis=1, keepdims=True))


def kernel(support, query, conv_w1, conv_b1, conv_w2, conv_b2, conv_w3,
           conv_b3, conv_w4, conv_b4, bn_g1, bn_be1, bn_g2, bn_be2, bn_g3,
           bn_be3, bn_g4, bn_be4, att_w, att_b, gru_wi0, gru_wh0, gru_bi0,
           gru_bh0, gru_wi1, gru_wh1, gru_bi1, gru_bh1, trans_w, trans_b):
    del trans_b

    imgs = jnp.concatenate(
        [support.reshape(25, 224, 224), query.reshape(15, 224, 224)], axis=0)
    xpad = jnp.pad(imgs, ((0, 0), (1, 1), (1, 1)))
    par = xpad.reshape(40, 113, 2, 113, 2).transpose(0, 2, 4, 1, 3)
    taps = [par[:, dy & 1, dx & 1, dy // 2:dy // 2 + 112, dx // 2:dx // 2 + 112]
            for dy in range(3) for dx in range(3)]
    x9 = jnp.stack(taps, axis=-1).reshape(40, 12544, 9)

    def stp(g, be, b):
        s = g * _BN_INV
        return jnp.stack([s, b * s + be])

    w9 = conv_w1.reshape(32, 9).T
    st1 = stp(bn_g1, bn_be1, conv_b1)
    w2r = conv_w2.transpose(2, 3, 1, 0).reshape(9, 32, 64)
    st2 = stp(bn_g2, bn_be2, conv_b2)
    w3r = conv_w3.transpose(2, 3, 1, 0).reshape(9, 64, 128)
    st3 = stp(bn_g3, bn_be3, conv_b3)
    w4r = conv_w4.transpose(2, 3, 1, 0).reshape(9, 128, 256)
    st4 = stp(bn_g4, bn_be4, conv_b4)
    attwT = att_w.T
    attb2 = att_b[None, :]
    wA = gru_wi0[:, :256].T
    wB = gru_wi0[:, 256:].T
    bi0 = gru_bi0[None, :]

    full = lambda shape: pl.BlockSpec(shape, lambda c, i: (0,) * len(shape))
    img = lambda nd: (lambda c, i: (c * 20 + i,) + (0,) * (nd - 1))
    attp, pA, pB = pl.pallas_call(
        _enc_body,
        grid=(2, 20),
        in_specs=[
            pl.BlockSpec((1, 12544, 9), img(3)),
            full((9, 32)), full((2, 32)),
            full((9, 32, 64)), full((2, 64)),
            full((9, 64, 128)), full((2, 128)),
            full((9, 128, 256)), full((2, 256)),
            full((256, 64)), full((1, 64)),
            full((256, 384)), full((256, 384)), full((1, 384)),
        ],
        out_specs=[
            pl.BlockSpec((1, 49, 64), img(3)),
            pl.BlockSpec((1, 49, 384), img(3)),
            pl.BlockSpec((1, 49, 384), img(3)),
        ],
        out_shape=[
            jax.ShapeDtypeStruct((40, 49, 64), _F32),
            jax.ShapeDtypeStruct((40, 49, 384), _F32),
            jax.ShapeDtypeStruct((40, 49, 384), _F32),
        ],
        compiler_params=pltpu.CompilerParams(
            dimension_semantics=("parallel", "arbitrary")),
        name="encoder",
    )(x9, w9, st1, w2r, st2, w3r, st3, w4r, st4, attwT, attb2, wA, wB, bi0)

    atth, attq = attp[:25], attp[25:]
    qfB = pB[25:]
    hfAa = jnp.concatenate(
        [pA[:25], jnp.ones((25, 49, 1), _F32)], axis=-1)

    pidx = lambda c, j: jnp.minimum(c * 188 + j, 374)
    gi = pl.pallas_call(
        _att_body,
        grid=(2, 188),
        in_specs=[
            pl.BlockSpec((1, 49, 64), lambda c, j: (pidx(c, j) // 25, 0, 0)),
            pl.BlockSpec((1, 49, 64), lambda c, j: (pidx(c, j) % 25, 0, 0)),
            pl.BlockSpec((1, 49, 385), lambda c, j: (pidx(c, j) % 25, 0, 0)),
            pl.BlockSpec((1, 49, 384), lambda c, j: (pidx(c, j) // 25, 0, 0)),
        ],
        out_specs=pl.BlockSpec((49, 1, 1, 384),
                               lambda c, j: (0, pidx(c, j), 0, 0)),
        out_shape=jax.ShapeDtypeStruct((49, 384, 1, 384), _F32),
        compiler_params=pltpu.CompilerParams(
            dimension_semantics=("parallel", "arbitrary")),
        name="attention",
    )(attq, atth, hfAa, qfB)

    gi_t = gi.reshape(49, 384, 384)

    wh0T = gru_wh0.T
    wi1T = gru_wi1.T
    wh1T = gru_wh1.T
    bh0 = gru_bh0[None, :]
    bi1 = gru_bi1[None, :]
    bh1 = gru_bh1[None, :]
    tw = trans_w

    wspec = lambda shape: pl.BlockSpec(shape, lambda c, t: (0,) * len(shape))
    h_out = pl.pallas_call(
        _gru_body,
        grid=(2, 49),
        in_specs=[
            pl.BlockSpec((1, 192, 384), lambda c, t: (t, c, 0)),
            wspec((128, 384)), wspec((1, 384)),
            wspec((128, 384)), wspec((1, 384)),
            wspec((128, 384)), wspec((1, 384)),
            wspec((1, 128)),
        ],
        out_specs=pl.BlockSpec((1, 1, 192), lambda c, t: (c, 0, 0)),
        out_shape=jax.ShapeDtypeStruct((2, 1, 192), _F32),
        scratch_shapes=[pltpu.VMEM((192, 128), _F32),
                        pltpu.VMEM((192, 128), _F32)],
        compiler_params=pltpu.CompilerParams(
            dimension_semantics=("parallel", "arbitrary")),
        name="gru",
    )(gi_t, wh0T, bh0, wi1T, bi1, wh1T, bh1, tw)

    Lq = h_out.reshape(384)[:375].reshape(15, 25)
    return pl.pallas_call(
        _head_body,
        out_shape=jax.ShapeDtypeStruct((15, 5), _F32),
        name="head",
    )(Lq)


# speedup vs baseline: 1.6365x; 1.2009x over previous
import jax
import jax.numpy as jnp
import numpy as np
from jax.experimental import pallas as pl
from jax.experimental.pallas import tpu as pltpu

_BN_INV = float(1.0 / np.sqrt(1.0 + 1e-5))
_F32 = jnp.float32


def _pool(x):
    A, B, C = x.shape
    x = x.reshape(A // 2, 2, B, C)
    x = jnp.maximum(x[:, 0], x[:, 1])
    x = jnp.swapaxes(x, 0, 1)
    x = x.reshape(B // 2, 2, A // 2, C)
    return jnp.maximum(x[:, 0], x[:, 1])


def _conv_block(h, w_ref, st_ref, H, Cin, Cout, swapped):
    hp = jnp.pad(h, ((1, 1), (1, 1), (0, 0)))
    acc = None
    for dy in range(3):
        for dx in range(3):
            a, b = (dx, dy) if swapped else (dy, dx)
            sl = hp[a:a + H, b:b + H, :].reshape(H * H, Cin)
            p = jnp.dot(sl, w_ref[dy * 3 + dx], preferred_element_type=_F32)
            acc = p if acc is None else acc + p
    y = jnp.maximum(acc * st_ref[0:1, :] + st_ref[1:2, :], 0.0)
    return _pool(y.reshape(H, H, Cout))


def _enc_body(x9_ref, w9_ref, st1_ref, w2_ref, st2_ref, w3_ref, st3_ref,
              w4_ref, st4_ref, attw_ref, attb_ref, wA_ref, wB_ref, bi0_ref,
              attp_ref, pA_ref, pB_ref):
    x9 = x9_ref[0]
    c = jnp.dot(x9, w9_ref[...], preferred_element_type=_F32)
    c = jnp.maximum(c * st1_ref[0:1, :] + st1_ref[1:2, :], 0.0)
    h = _pool(c.reshape(112, 112, 32))
    h = _conv_block(h, w2_ref, st2_ref, 56, 32, 64, True)
    h = _conv_block(h, w3_ref, st3_ref, 28, 64, 128, False)
    h = _conv_block(h, w4_ref, st4_ref, 14, 128, 256, True)
    feat = h.reshape(49, 256)
    attp_ref[0] = jnp.dot(feat, attw_ref[...],
                          preferred_element_type=_F32) + attb_ref[...]
    pA_ref[0] = jnp.dot(feat, wA_ref[...], preferred_element_type=_F32)
    pB_ref[0] = jnp.dot(feat, wB_ref[...],
                        preferred_element_type=_F32) + bi0_ref[...]


def _att_body(wq_ref, wh_ref, hfAa_ref, qfB_ref, out_ref):
    wq = wq_ref[0]
    qfB = qfB_ref[0]
    for k in range(5):
        wh = wh_ref[k]
        prod = wq[:, None, :] * wh[None, :, :]
        scores = jnp.sum(jnp.tanh(prod), axis=-1)
        e = jnp.exp(scores)
        nd = jnp.dot(e, hfAa_ref[k], preferred_element_type=_F32)
        out_ref[:, k, 0] = nd[:, :384] / nd[:, 384:385] + qfB


def _gru_body(gi_ref, wh0_ref, bh0_ref, wi1_ref, bi1_ref, wh1_ref, bh1_ref,
              tw_ref, out_ref, h0_ref, h1_ref):
    t = pl.program_id(0)

    @pl.when(t == 0)
    def _():
        h0_ref[...] = jnp.zeros_like(h0_ref)
        h1_ref[...] = jnp.zeros_like(h1_ref)

    gi = gi_ref[0]
    h0 = h0_ref[...]
    gh = jnp.dot(h0, wh0_ref[...], preferred_element_type=_F32) + bh0_ref[...]
    r = jax.nn.sigmoid(gi[:, 0:128] + gh[:, 0:128])
    z = jax.nn.sigmoid(gi[:, 128:256] + gh[:, 128:256])
    n = jnp.tanh(gi[:, 256:384] + r * gh[:, 256:384])
    h0 = (1.0 - z) * n + z * h0
    h0_ref[...] = h0

    gi1 = jnp.dot(h0, wi1_ref[...], preferred_element_type=_F32) + bi1_ref[...]
    h1 = h1_ref[...]
    gh1 = jnp.dot(h1, wh1_ref[...], preferred_element_type=_F32) + bh1_ref[...]
    r1 = jax.nn.sigmoid(gi1[:, 0:128] + gh1[:, 0:128])
    z1 = jax.nn.sigmoid(gi1[:, 128:256] + gh1[:, 128:256])
    n1 = jnp.tanh(gi1[:, 256:384] + r1 * gh1[:, 256:384])
    h1 = (1.0 - z1) * n1 + z1 * h1
    h1_ref[...] = h1

    @pl.when(t == 48)
    def _():
        out_ref[...] = jax.lax.dot_general(
            tw_ref[...], h1, (((1,), (1,)), ((), ())),
            preferred_element_type=_F32)


def _head_body(L_ref, out_ref):
    L = L_ref[...]
    u = jax.lax.broadcasted_iota(jnp.int32, (25, 5), 0)
    c = jax.lax.broadcasted_iota(jnp.int32, (25, 5), 1)
    ksum = jnp.where(u // 5 == c, 1.0, 0.0).astype(_F32)
    S = jnp.dot(L, ksum, preferred_element_type=_F32)
    m = jnp.max(S, axis=1, keepdims=True)
    e = jnp.exp(S - m)
    out_ref[...] = (S - m) - jnp.log(jnp.sum(e, axis=1, keepdims=True))


def kernel(support, query, conv_w1, conv_b1, conv_w2, conv_b2, conv_w3,
           conv_b3, conv_w4, conv_b4, bn_g1, bn_be1, bn_g2, bn_be2, bn_g3,
           bn_be3, bn_g4, bn_be4, att_w, att_b, gru_wi0, gru_wh0, gru_bi0,
           gru_bh0, gru_wi1, gru_wh1, gru_bi1, gru_bh1, trans_w, trans_b):
    del trans_b

    imgs = jnp.concatenate(
        [support.reshape(25, 224, 224), query.reshape(15, 224, 224)], axis=0)
    xpad = jnp.pad(imgs, ((0, 0), (1, 1), (1, 1)))
    par = xpad.reshape(40, 113, 2, 113, 2).transpose(0, 2, 4, 1, 3)
    taps = [par[:, dy & 1, dx & 1, dy // 2:dy // 2 + 112, dx // 2:dx // 2 + 112]
            for dy in range(3) for dx in range(3)]
    x9 = jnp.stack(taps, axis=-1).reshape(40, 12544, 9)

    def stp(g, be, b):
        s = g * _BN_INV
        return jnp.stack([s, b * s + be])

    w9 = conv_w1.reshape(32, 9).T
    st1 = stp(bn_g1, bn_be1, conv_b1)
    w2r = conv_w2.transpose(2, 3, 1, 0).reshape(9, 32, 64)
    st2 = stp(bn_g2, bn_be2, conv_b2)
    w3r = conv_w3.transpose(2, 3, 1, 0).reshape(9, 64, 128)
    st3 = stp(bn_g3, bn_be3, conv_b3)
    w4r = conv_w4.transpose(2, 3, 1, 0).reshape(9, 128, 256)
    st4 = stp(bn_g4, bn_be4, conv_b4)
    attwT = att_w.T
    attb2 = att_b[None, :]
    wA = gru_wi0[:, :256].T
    wB = gru_wi0[:, 256:].T
    bi0 = gru_bi0[None, :]

    full = lambda shape: pl.BlockSpec(shape, lambda c, i: (0,) * len(shape))
    img = lambda nd: (lambda c, i: (c * 20 + i,) + (0,) * (nd - 1))
    attp, pA, pB = pl.pallas_call(
        _enc_body,
        grid=(2, 20),
        in_specs=[
            pl.BlockSpec((1, 12544, 9), img(3)),
            full((9, 32)), full((2, 32)),
            full((9, 32, 64)), full((2, 64)),
            full((9, 64, 128)), full((2, 128)),
            full((9, 128, 256)), full((2, 256)),
            full((256, 64)), full((1, 64)),
            full((256, 384)), full((256, 384)), full((1, 384)),
        ],
        out_specs=[
            pl.BlockSpec((1, 49, 64), img(3)),
            pl.BlockSpec((1, 49, 384), img(3)),
            pl.BlockSpec((1, 49, 384), img(3)),
        ],
        out_shape=[
            jax.ShapeDtypeStruct((40, 49, 64), _F32),
            jax.ShapeDtypeStruct((40, 49, 384), _F32),
            jax.ShapeDtypeStruct((40, 49, 384), _F32),
        ],
        compiler_params=pltpu.CompilerParams(
            dimension_semantics=("parallel", "arbitrary")),
        name="encoder",
    )(x9, w9, st1, w2r, st2, w3r, st3, w4r, st4, attwT, attb2, wA, wB, bi0)

    atth, attq = attp[:25], attp[25:]
    qfB = pB[25:]
    hfAa = jnp.concatenate(
        [pA[:25], jnp.ones((25, 49, 1), _F32)], axis=-1)

    gi = pl.pallas_call(
        _att_body,
        grid=(15, 5),
        in_specs=[
            pl.BlockSpec((1, 49, 64), lambda q, sb: (q, 0, 0)),
            pl.BlockSpec((5, 49, 64), lambda q, sb: (sb, 0, 0)),
            pl.BlockSpec((5, 49, 385), lambda q, sb: (sb, 0, 0)),
            pl.BlockSpec((1, 49, 384), lambda q, sb: (q, 0, 0)),
        ],
        out_specs=pl.BlockSpec((49, 5, 1, 384),
                               lambda q, sb: (0, q * 5 + sb, 0, 0)),
        out_shape=jax.ShapeDtypeStruct((49, 375, 1, 384), _F32),
        compiler_params=pltpu.CompilerParams(
            dimension_semantics=("parallel", "arbitrary")),
        name="attention",
    )(attq, atth, hfAa, qfB)

    gi_t = gi.reshape(49, 375, 384)

    wh0T = gru_wh0.T
    wi1T = gru_wi1.T
    wh1T = gru_wh1.T
    bh0 = gru_bh0[None, :]
    bi1 = gru_bi1[None, :]
    bh1 = gru_bh1[None, :]
    tw = trans_w

    wspec = lambda shape: pl.BlockSpec(shape, lambda t: (0,) * len(shape))
    h_out = pl.pallas_call(
        _gru_body,
        grid=(49,),
        in_specs=[
            pl.BlockSpec((1, 375, 384), lambda t: (t, 0, 0)),
            wspec((128, 384)), wspec((1, 384)),
            wspec((128, 384)), wspec((1, 384)),
            wspec((128, 384)), wspec((1, 384)),
            wspec((1, 128)),
        ],
        out_specs=pl.BlockSpec((1, 375), lambda t: (0, 0)),
        out_shape=jax.ShapeDtypeStruct((1, 375), _F32),
        scratch_shapes=[pltpu.VMEM((375, 128), _F32),
                        pltpu.VMEM((375, 128), _F32)],
        compiler_params=pltpu.CompilerParams(
            dimension_semantics=("arbitrary",)),
        name="gru",
    )(gi_t, wh0T, bh0, wi1T, bi1, wh1T, bh1, tw)

    Lq = h_out.reshape(15, 25)
    return pl.pallas_call(
        _head_body,
        out_shape=jax.ShapeDtypeStruct((15, 5), _F32),
        name="head",
    )(Lq)


# speedup vs baseline: 1.6886x; 1.0318x over previous
import jax
import jax.numpy as jnp
import numpy as np
from jax.experimental import pallas as pl
from jax.experimental.pallas import tpu as pltpu

_BN_INV = float(1.0 / np.sqrt(1.0 + 1e-5))
_F32 = jnp.float32


def _pool(x):
    A, B, C = x.shape
    x = x.reshape(A // 2, 2, B, C)
    x = jnp.maximum(x[:, 0], x[:, 1])
    x = jnp.swapaxes(x, 0, 1)
    x = x.reshape(B // 2, 2, A // 2, C)
    return jnp.maximum(x[:, 0], x[:, 1])


def _conv_block(h, w_ref, st_ref, H, Cin, Cout, swapped):
    hp = jnp.pad(h, ((1, 1), (1, 1), (0, 0)))
    acc = None
    for dy in range(3):
        for dx in range(3):
            a, b = (dx, dy) if swapped else (dy, dx)
            sl = hp[a:a + H, b:b + H, :].reshape(H * H, Cin)
            p = jnp.dot(sl, w_ref[dy * 3 + dx],
                        preferred_element_type=_F32)
            acc = p if acc is None else acc + p
    y = jnp.maximum(acc * st_ref[0:1, :] + st_ref[1:2, :], 0.0)
    return _pool(y.reshape(H, H, Cout))


def _enc_body(x9_ref, w9_ref, st1_ref, w2_ref, st2_ref, w3_ref, st3_ref,
              w4_ref, st4_ref, attw_ref, attb_ref, wA_ref, wB_ref, bi0_ref,
              attp_ref, pA_ref, pB_ref):
    x9 = x9_ref[0]
    c = jax.lax.dot_general(x9, w9_ref[...], (((0,), (0,)), ((), ())),
                            preferred_element_type=_F32)
    c = jnp.maximum(c * st1_ref[0:1, :] + st1_ref[1:2, :], 0.0)
    h = _pool(c.reshape(112, 112, 32))
    h = _conv_block(h, w2_ref, st2_ref, 56, 32, 64, True)
    h = _conv_block(h, w3_ref, st3_ref, 28, 64, 128, False)
    h = _conv_block(h, w4_ref, st4_ref, 14, 128, 256, True)
    feat = h.reshape(49, 256)
    attp_ref[0] = jnp.dot(feat, attw_ref[...],
                          preferred_element_type=_F32) + attb_ref[...]
    pA_ref[0] = jnp.dot(feat, wA_ref[...], preferred_element_type=_F32)
    pB_ref[0] = jnp.dot(feat, wB_ref[...],
                        preferred_element_type=_F32) + bi0_ref[...]


def _att_body(wq_ref, wh_ref, hfAa_ref, qfB_ref, out_ref):
    wq = wq_ref[0]
    qfB = qfB_ref[0]
    for k in range(5):
        wh = wh_ref[k]
        prod = wq[:, None, :] * wh[None, :, :]
        scores = jnp.sum(jnp.tanh(prod), axis=-1)
        e = jnp.exp(scores)
        nd = jnp.dot(e, hfAa_ref[k], preferred_element_type=_F32)
        out_ref[:, k, 0] = nd[:, :384] / nd[:, 384:385] + qfB


def _gru_body(gi_ref, wh0_ref, bh0_ref, wi1_ref, bi1_ref, wh1_ref, bh1_ref,
              tw_ref, out_ref, h0_ref, h1_ref):
    t = pl.program_id(0)

    @pl.when(t == 0)
    def _():
        h0_ref[...] = jnp.zeros_like(h0_ref)
        h1_ref[...] = jnp.zeros_like(h1_ref)

    gi = gi_ref[0]
    h0 = h0_ref[...]
    gh = jnp.dot(h0, wh0_ref[...], preferred_element_type=_F32) + bh0_ref[...]
    r = jax.nn.sigmoid(gi[:, 0:128] + gh[:, 0:128])
    z = jax.nn.sigmoid(gi[:, 128:256] + gh[:, 128:256])
    n = jnp.tanh(gi[:, 256:384] + r * gh[:, 256:384])
    h0 = (1.0 - z) * n + z * h0
    h0_ref[...] = h0

    gi1 = jnp.dot(h0, wi1_ref[...], preferred_element_type=_F32) + bi1_ref[...]
    h1 = h1_ref[...]
    gh1 = jnp.dot(h1, wh1_ref[...], preferred_element_type=_F32) + bh1_ref[...]
    r1 = jax.nn.sigmoid(gi1[:, 0:128] + gh1[:, 0:128])
    z1 = jax.nn.sigmoid(gi1[:, 128:256] + gh1[:, 128:256])
    n1 = jnp.tanh(gi1[:, 256:384] + r1 * gh1[:, 256:384])
    h1 = (1.0 - z1) * n1 + z1 * h1
    h1_ref[...] = h1

    @pl.when(t == 48)
    def _():
        out_ref[...] = jax.lax.dot_general(
            tw_ref[...], h1, (((1,), (1,)), ((), ())),
            preferred_element_type=_F32)


def _head_body(L_ref, out_ref):
    L = L_ref[...]
    u = jax.lax.broadcasted_iota(jnp.int32, (25, 5), 0)
    c = jax.lax.broadcasted_iota(jnp.int32, (25, 5), 1)
    ksum = jnp.where(u // 5 == c, 1.0, 0.0).astype(_F32)
    S = jnp.dot(L, ksum, preferred_element_type=_F32)
    m = jnp.max(S, axis=1, keepdims=True)
    e = jnp.exp(S - m)
    out_ref[...] = (S - m) - jnp.log(jnp.sum(e, axis=1, keepdims=True))


def kernel(support, query, conv_w1, conv_b1, conv_w2, conv_b2, conv_w3,
           conv_b3, conv_w4, conv_b4, bn_g1, bn_be1, bn_g2, bn_be2, bn_g3,
           bn_be3, bn_g4, bn_be4, att_w, att_b, gru_wi0, gru_wh0, gru_bi0,
           gru_bh0, gru_wi1, gru_wh1, gru_bi1, gru_bh1, trans_w, trans_b):
    del trans_b

    imgs = jnp.concatenate(
        [support.reshape(25, 224, 224), query.reshape(15, 224, 224)], axis=0)
    xpad = jnp.pad(imgs, ((0, 0), (1, 1), (1, 1)))
    par = xpad.reshape(40, 113, 2, 113, 2).transpose(0, 2, 4, 1, 3)
    taps = [par[:, dy & 1, dx & 1, dy // 2:dy // 2 + 112, dx // 2:dx // 2 + 112]
            for dy in range(3) for dx in range(3)]
    x9 = jnp.stack(taps, axis=1).reshape(40, 9, 12544)

    def stp(g, be, b):
        s = g * _BN_INV
        return jnp.stack([s, b * s + be])

    w9 = conv_w1.reshape(32, 9).T
    st1 = stp(bn_g1, bn_be1, conv_b1)
    w2r = conv_w2.transpose(2, 3, 1, 0).reshape(9, 32, 64)
    st2 = stp(bn_g2, bn_be2, conv_b2)
    w3r = conv_w3.transpose(2, 3, 1, 0).reshape(9, 64, 128)
    st3 = stp(bn_g3, bn_be3, conv_b3)
    w4r = conv_w4.transpose(2, 3, 1, 0).reshape(9, 128, 256)
    st4 = stp(bn_g4, bn_be4, conv_b4)
    attwT = att_w.T
    attb2 = att_b[None, :]
    wA = gru_wi0[:, :256].T
    wB = gru_wi0[:, 256:].T
    bi0 = gru_bi0[None, :]

    full = lambda shape: pl.BlockSpec(shape, lambda c, i: (0,) * len(shape))
    img = lambda nd: (lambda c, i: (c * 20 + i,) + (0,) * (nd - 1))
    attp, pA, pB = pl.pallas_call(
        _enc_body,
        grid=(2, 20),
        in_specs=[
            pl.BlockSpec((1, 9, 12544), img(3)),
            full((9, 32)), full((2, 32)),
            full((9, 32, 64)), full((2, 64)),
            full((9, 64, 128)), full((2, 128)),
            full((9, 128, 256)), full((2, 256)),
            full((256, 64)), full((1, 64)),
            full((256, 384)), full((256, 384)), full((1, 384)),
        ],
        out_specs=[
            pl.BlockSpec((1, 49, 64), img(3)),
            pl.BlockSpec((1, 49, 384), img(3)),
            pl.BlockSpec((1, 49, 384), img(3)),
        ],
        out_shape=[
            jax.ShapeDtypeStruct((40, 49, 64), _F32),
            jax.ShapeDtypeStruct((40, 49, 384), _F32),
            jax.ShapeDtypeStruct((40, 49, 384), _F32),
        ],
        compiler_params=pltpu.CompilerParams(
            dimension_semantics=("parallel", "arbitrary")),
        name="encoder",
    )(x9, w9, st1, w2r, st2, w3r, st3, w4r, st4, attwT, attb2, wA, wB, bi0)

    atth, attq = attp[:25], attp[25:]
    qfB = pB[25:]
    hfAa = jnp.concatenate(
        [pA[:25], jnp.ones((25, 49, 1), _F32)], axis=-1)

    gi = pl.pallas_call(
        _att_body,
        grid=(15, 5),
        in_specs=[
            pl.BlockSpec((1, 49, 64), lambda q, sb: (q, 0, 0)),
            pl.BlockSpec((5, 49, 64), lambda q, sb: (sb, 0, 0)),
            pl.BlockSpec((5, 49, 385), lambda q, sb: (sb, 0, 0)),
            pl.BlockSpec((1, 49, 384), lambda q, sb: (q, 0, 0)),
        ],
        out_specs=pl.BlockSpec((49, 5, 1, 384),
                               lambda q, sb: (0, q * 5 + sb, 0, 0)),
        out_shape=jax.ShapeDtypeStruct((49, 375, 1, 384), _F32),
        compiler_params=pltpu.CompilerParams(
            dimension_semantics=("parallel", "arbitrary")),
        name="attention",
    )(attq, atth, hfAa, qfB)

    gi_t = gi.reshape(49, 375, 384)

    wh0T = gru_wh0.T
    wi1T = gru_wi1.T
    wh1T = gru_wh1.T
    bh0 = gru_bh0[None, :]
    bi1 = gru_bi1[None, :]
    bh1 = gru_bh1[None, :]
    tw = trans_w

    wspec = lambda shape: pl.BlockSpec(shape, lambda t: (0,) * len(shape))
    h_out = pl.pallas_call(
        _gru_body,
        grid=(49,),
        in_specs=[
            pl.BlockSpec((1, 375, 384), lambda t: (t, 0, 0)),
            wspec((128, 384)), wspec((1, 384)),
            wspec((128, 384)), wspec((1, 384)),
            wspec((128, 384)), wspec((1, 384)),
            wspec((1, 128)),
        ],
        out_specs=pl.BlockSpec((1, 375), lambda t: (0, 0)),
        out_shape=jax.ShapeDtypeStruct((1, 375), _F32),
        scratch_shapes=[pltpu.VMEM((375, 128), _F32),
                        pltpu.VMEM((375, 128), _F32)],
        compiler_params=pltpu.CompilerParams(
            dimension_semantics=("arbitrary",)),
        name="gru",
    )(gi_t, wh0T, bh0, wi1T, bi1, wh1T, bh1, tw)

    Lq = h_out.reshape(15, 25)
    return pl.pallas_call(
        _head_body,
        out_shape=jax.ShapeDtypeStruct((15, 5), _F32),
        name="head",
    )(Lq)


# speedup vs baseline: 1.7048x; 1.0096x over previous
import jax
import jax.numpy as jnp
import numpy as np
from jax.experimental import pallas as pl
from jax.experimental.pallas import tpu as pltpu

_BN_INV = float(1.0 / np.sqrt(1.0 + 1e-5))
_F32 = jnp.float32


def _pool(x):
    A, B, C = x.shape
    x = x.reshape(A // 2, 2, B, C)
    x = jnp.maximum(x[:, 0], x[:, 1])
    x = jnp.swapaxes(x, 0, 1)
    x = x.reshape(B // 2, 2, A // 2, C)
    return jnp.maximum(x[:, 0], x[:, 1])


def _conv_block(h, w_ref, st_ref, H, Cin, Cout, swapped):
    hp = jnp.pad(h, ((1, 1), (1, 1), (0, 0)))
    acc = None
    for dy in range(3):
        for dx in range(3):
            a, b = (dx, dy) if swapped else (dy, dx)
            sl = hp[a:a + H, b:b + H, :].reshape(H * H, Cin)
            p = jnp.dot(sl, w_ref[dy * 3 + dx],
                        preferred_element_type=_F32)
            acc = p if acc is None else acc + p
    y = jnp.maximum(acc * st_ref[0:1, :] + st_ref[1:2, :], 0.0)
    return _pool(y.reshape(H, H, Cout))


def _enc_body(x9_ref, w9_ref, st1_ref, w2_ref, st2_ref, w3_ref, st3_ref,
              w4_ref, st4_ref, attw_ref, attb_ref, wA_ref, wB_ref, bi0_ref,
              attp_ref, pA_ref, pB_ref):
    x9 = x9_ref[0]
    c = jax.lax.dot_general(x9, w9_ref[...], (((0,), (0,)), ((), ())),
                            preferred_element_type=_F32)
    c = jnp.maximum(c * st1_ref[0:1, :] + st1_ref[1:2, :], 0.0)
    h = _pool(c.reshape(112, 112, 32))
    h = _conv_block(h, w2_ref, st2_ref, 56, 32, 64, True)
    h = _conv_block(h, w3_ref, st3_ref, 28, 64, 128, False)
    h = _conv_block(h, w4_ref, st4_ref, 14, 128, 256, True)
    feat = h.reshape(49, 256)
    attp_ref[0] = jnp.dot(feat, attw_ref[...],
                          preferred_element_type=_F32) + attb_ref[...]
    pA_ref[0, :, 0:384] = jnp.dot(feat, wA_ref[...],
                                  preferred_element_type=_F32)
    pA_ref[0, :, 384:385] = jnp.ones((49, 1), _F32)
    pB_ref[0] = jnp.dot(feat, wB_ref[...],
                        preferred_element_type=_F32) + bi0_ref[...]


def _att_body(wq_ref, wh_ref, hfAa_ref, qfB_ref, out_ref):
    wq = wq_ref[0]
    qfB = qfB_ref[0]
    for k in range(5):
        wh = wh_ref[k]
        prod = wq[:, None, :] * wh[None, :, :]
        scores = jnp.sum(jnp.tanh(prod), axis=-1)
        e = jnp.exp(scores)
        nd = jnp.dot(e, hfAa_ref[k], preferred_element_type=_F32)
        out_ref[:, k, 0] = nd[:, :384] / nd[:, 384:385] + qfB


def _gru_body(gi_ref, wh0_ref, bh0_ref, wi1_ref, bi1_ref, wh1_ref, bh1_ref,
              tw_ref, out_ref, h0_ref, h1_ref):
    t = pl.program_id(0)

    @pl.when(t == 0)
    def _():
        h0_ref[...] = jnp.zeros_like(h0_ref)
        h1_ref[...] = jnp.zeros_like(h1_ref)

    gi = gi_ref[0]
    h0 = h0_ref[...]
    gh = jnp.dot(h0, wh0_ref[...], preferred_element_type=_F32) + bh0_ref[...]
    r = jax.nn.sigmoid(gi[:, 0:128] + gh[:, 0:128])
    z = jax.nn.sigmoid(gi[:, 128:256] + gh[:, 128:256])
    n = jnp.tanh(gi[:, 256:384] + r * gh[:, 256:384])
    h0 = (1.0 - z) * n + z * h0
    h0_ref[...] = h0

    gi1 = jnp.dot(h0, wi1_ref[...], preferred_element_type=_F32) + bi1_ref[...]
    h1 = h1_ref[...]
    gh1 = jnp.dot(h1, wh1_ref[...], preferred_element_type=_F32) + bh1_ref[...]
    r1 = jax.nn.sigmoid(gi1[:, 0:128] + gh1[:, 0:128])
    z1 = jax.nn.sigmoid(gi1[:, 128:256] + gh1[:, 128:256])
    n1 = jnp.tanh(gi1[:, 256:384] + r1 * gh1[:, 256:384])
    h1 = (1.0 - z1) * n1 + z1 * h1
    h1_ref[...] = h1

    @pl.when(t == 48)
    def _():
        out_ref[...] = jax.lax.dot_general(
            tw_ref[...], h1, (((1,), (1,)), ((), ())),
            preferred_element_type=_F32)


def _head_body(L_ref, out_ref):
    L = L_ref[...]
    u = jax.lax.broadcasted_iota(jnp.int32, (25, 5), 0)
    c = jax.lax.broadcasted_iota(jnp.int32, (25, 5), 1)
    ksum = jnp.where(u // 5 == c, 1.0, 0.0).astype(_F32)
    S = jnp.dot(L, ksum, preferred_element_type=_F32)
    m = jnp.max(S, axis=1, keepdims=True)
    e = jnp.exp(S - m)
    out_ref[...] = (S - m) - jnp.log(jnp.sum(e, axis=1, keepdims=True))


def kernel(support, query, conv_w1, conv_b1, conv_w2, conv_b2, conv_w3,
           conv_b3, conv_w4, conv_b4, bn_g1, bn_be1, bn_g2, bn_be2, bn_g3,
           bn_be3, bn_g4, bn_be4, att_w, att_b, gru_wi0, gru_wh0, gru_bi0,
           gru_bh0, gru_wi1, gru_wh1, gru_bi1, gru_bh1, trans_w, trans_b):
    del trans_b

    imgs = jnp.concatenate(
        [support.reshape(25, 224, 224), query.reshape(15, 224, 224)], axis=0)
    xpad = jnp.pad(imgs, ((0, 0), (1, 1), (1, 1)))
    par = xpad.reshape(40, 113, 2, 113, 2).transpose(0, 2, 4, 1, 3)
    taps = [par[:, dy & 1, dx & 1, dy // 2:dy // 2 + 112, dx // 2:dx // 2 + 112]
            for dy in range(3) for dx in range(3)]
    x9 = jnp.stack(taps, axis=1).reshape(40, 9, 12544)

    def stp(g, be, b):
        s = g * _BN_INV
        return jnp.stack([s, b * s + be])

    w9 = conv_w1.reshape(32, 9).T
    st1 = stp(bn_g1, bn_be1, conv_b1)
    w2r = conv_w2.transpose(2, 3, 1, 0).reshape(9, 32, 64)
    st2 = stp(bn_g2, bn_be2, conv_b2)
    w3r = conv_w3.transpose(2, 3, 1, 0).reshape(9, 64, 128)
    st3 = stp(bn_g3, bn_be3, conv_b3)
    w4r = conv_w4.transpose(2, 3, 1, 0).reshape(9, 128, 256)
    st4 = stp(bn_g4, bn_be4, conv_b4)
    attwT = att_w.T
    attb2 = att_b[None, :]
    wA = gru_wi0[:, :256].T
    wB = gru_wi0[:, 256:].T
    bi0 = gru_bi0[None, :]

    full = lambda shape: pl.BlockSpec(shape, lambda c, i: (0,) * len(shape))
    img = lambda nd: (lambda c, i: (c * 20 + i,) + (0,) * (nd - 1))
    attp, pA, pB = pl.pallas_call(
        _enc_body,
        grid=(2, 20),
        in_specs=[
            pl.BlockSpec((1, 9, 12544), img(3)),
            full((9, 32)), full((2, 32)),
            full((9, 32, 64)), full((2, 64)),
            full((9, 64, 128)), full((2, 128)),
            full((9, 128, 256)), full((2, 256)),
            full((256, 64)), full((1, 64)),
            full((256, 384)), full((256, 384)), full((1, 384)),
        ],
        out_specs=[
            pl.BlockSpec((1, 49, 64), img(3)),
            pl.BlockSpec((1, 49, 385), img(3)),
            pl.BlockSpec((1, 49, 384), img(3)),
        ],
        out_shape=[
            jax.ShapeDtypeStruct((40, 49, 64), _F32),
            jax.ShapeDtypeStruct((40, 49, 385), _F32),
            jax.ShapeDtypeStruct((40, 49, 384), _F32),
        ],
        compiler_params=pltpu.CompilerParams(
            dimension_semantics=("parallel", "arbitrary")),
        name="encoder",
    )(x9, w9, st1, w2r, st2, w3r, st3, w4r, st4, attwT, attb2, wA, wB, bi0)

    gi = pl.pallas_call(
        _att_body,
        grid=(15, 5),
        in_specs=[
            pl.BlockSpec((1, 49, 64), lambda q, sb: (25 + q, 0, 0)),
            pl.BlockSpec((5, 49, 64), lambda q, sb: (sb, 0, 0)),
            pl.BlockSpec((5, 49, 385), lambda q, sb: (sb, 0, 0)),
            pl.BlockSpec((1, 49, 384), lambda q, sb: (25 + q, 0, 0)),
        ],
        out_specs=pl.BlockSpec((49, 5, 1, 384),
                               lambda q, sb: (0, q * 5 + sb, 0, 0)),
        out_shape=jax.ShapeDtypeStruct((49, 375, 1, 384), _F32),
        compiler_params=pltpu.CompilerParams(
            dimension_semantics=("parallel", "arbitrary")),
        name="attention",
    )(attp, attp, pA, pB)

    gi_t = gi.reshape(49, 375, 384)

    wh0T = gru_wh0.T
    wi1T = gru_wi1.T
    wh1T = gru_wh1.T
    bh0 = gru_bh0[None, :]
    bi1 = gru_bi1[None, :]
    bh1 = gru_bh1[None, :]
    tw = trans_w

    wspec = lambda shape: pl.BlockSpec(shape, lambda t: (0,) * len(shape))
    h_out = pl.pallas_call(
        _gru_body,
        grid=(49,),
        in_specs=[
            pl.BlockSpec((1, 375, 384), lambda t: (t, 0, 0)),
            wspec((128, 384)), wspec((1, 384)),
            wspec((128, 384)), wspec((1, 384)),
            wspec((128, 384)), wspec((1, 384)),
            wspec((1, 128)),
        ],
        out_specs=pl.BlockSpec((1, 375), lambda t: (0, 0)),
        out_shape=jax.ShapeDtypeStruct((1, 375), _F32),
        scratch_shapes=[pltpu.VMEM((375, 128), _F32),
                        pltpu.VMEM((375, 128), _F32)],
        compiler_params=pltpu.CompilerParams(
            dimension_semantics=("arbitrary",)),
        name="gru",
    )(gi_t, wh0T, bh0, wi1T, bi1, wh1T, bh1, tw)

    Lq = h_out.reshape(15, 25)
    return pl.pallas_call(
        _head_body,
        out_shape=jax.ShapeDtypeStruct((15, 5), _F32),
        name="head",
    )(Lq)


# speedup vs baseline: 1.7445x; 1.0233x over previous
import jax
import jax.numpy as jnp
import numpy as np
from jax.experimental import pallas as pl
from jax.experimental.pallas import tpu as pltpu

_BN_INV = float(1.0 / np.sqrt(1.0 + 1e-5))
_F32 = jnp.float32


def _pool(x):
    A, B, C = x.shape
    x = x.reshape(A // 2, 2, B, C)
    x = jnp.maximum(x[:, 0], x[:, 1])
    x = jnp.swapaxes(x, 0, 1)
    x = x.reshape(B // 2, 2, A // 2, C)
    return jnp.maximum(x[:, 0], x[:, 1])


def _conv_block(h, w_ref, st_ref, H, Cin, Cout, swapped):
    hp = jnp.pad(h, ((1, 1), (1, 1), (0, 0)))
    acc = None
    for dy in range(3):
        for dx in range(3):
            a, b = (dx, dy) if swapped else (dy, dx)
            sl = hp[a:a + H, b:b + H, :].reshape(H * H, Cin)
            p = jnp.dot(sl, w_ref[dy * 3 + dx],
                        preferred_element_type=_F32)
            acc = p if acc is None else acc + p
    y = jnp.maximum(acc * st_ref[0:1, :] + st_ref[1:2, :], 0.0)
    return _pool(y.reshape(H, H, Cout))


def _enc_body(x9_ref, w9_ref, st1_ref, w2_ref, st2_ref, w3_ref, st3_ref,
              w4_ref, st4_ref, attw_ref, attb_ref, wA_ref, wB_ref, bi0_ref,
              attp_ref, pA_ref, pB_ref):
    x9 = x9_ref[0]
    c = jax.lax.dot_general(x9, w9_ref[...], (((0,), (0,)), ((), ())),
                            preferred_element_type=_F32)
    c = jnp.maximum(c * st1_ref[0:1, :] + st1_ref[1:2, :], 0.0)
    h = _pool(c.reshape(112, 112, 32))
    h = _conv_block(h, w2_ref, st2_ref, 56, 32, 64, True)
    h = _conv_block(h, w3_ref, st3_ref, 28, 64, 128, False)
    h = _conv_block(h, w4_ref, st4_ref, 14, 128, 256, True)
    feat = h.reshape(49, 256)
    attp_ref[0] = jnp.dot(feat, attw_ref[...],
                          preferred_element_type=_F32) + attb_ref[...]
    pA_ref[0, :, 0:384] = jnp.dot(feat, wA_ref[...],
                                  preferred_element_type=_F32)
    pA_ref[0, :, 384:385] = jnp.ones((49, 1), _F32)
    pB_ref[0] = jnp.dot(feat, wB_ref[...],
                        preferred_element_type=_F32) + bi0_ref[...]


def _att_body(wq_ref, wh_ref, hfAa_ref, qfB_ref, out_ref):
    wq = wq_ref[0]
    qfB = qfB_ref[0]
    for k in range(5):
        wh = wh_ref[k]
        prod = wq[:, None, :] * wh[None, :, :]
        scores = jnp.sum(jnp.tanh(prod), axis=-1)
        e = jnp.exp(scores)
        nd = jnp.dot(e, hfAa_ref[k], preferred_element_type=_F32)
        out_ref[:, k, 0] = nd[:, :384] / nd[:, 384:385] + qfB


def _gru_body(gi_ref, wh0_ref, bh0_ref, wi1_ref, bi1_ref, wh1_ref, bh1_ref,
              tw_ref, out_ref, h0_ref, h1_ref):
    tb = pl.program_id(0)

    @pl.when(tb == 0)
    def _():
        h0_ref[...] = jnp.zeros_like(h0_ref)
        h1_ref[...] = jnp.zeros_like(h1_ref)

    h0 = h0_ref[...]
    h1 = h1_ref[...]
    for k in range(7):
        gi = gi_ref[k]
        gh = (jnp.dot(h0, wh0_ref[...], preferred_element_type=_F32)
              + bh0_ref[...])
        r = jax.nn.sigmoid(gi[:, 0:128] + gh[:, 0:128])
        z = jax.nn.sigmoid(gi[:, 128:256] + gh[:, 128:256])
        n = jnp.tanh(gi[:, 256:384] + r * gh[:, 256:384])
        h0 = (1.0 - z) * n + z * h0

        gi1 = (jnp.dot(h0, wi1_ref[...], preferred_element_type=_F32)
               + bi1_ref[...])
        gh1 = (jnp.dot(h1, wh1_ref[...], preferred_element_type=_F32)
               + bh1_ref[...])
        r1 = jax.nn.sigmoid(gi1[:, 0:128] + gh1[:, 0:128])
        z1 = jax.nn.sigmoid(gi1[:, 128:256] + gh1[:, 128:256])
        n1 = jnp.tanh(gi1[:, 256:384] + r1 * gh1[:, 256:384])
        h1 = (1.0 - z1) * n1 + z1 * h1
    h0_ref[...] = h0
    h1_ref[...] = h1

    @pl.when(tb == 6)
    def _():
        out_ref[...] = jax.lax.dot_general(
            tw_ref[...], h1, (((1,), (1,)), ((), ())),
            preferred_element_type=_F32)


def _head_body(L_ref, out_ref):
    L = L_ref[...]
    u = jax.lax.broadcasted_iota(jnp.int32, (25, 5), 0)
    c = jax.lax.broadcasted_iota(jnp.int32, (25, 5), 1)
    ksum = jnp.where(u // 5 == c, 1.0, 0.0).astype(_F32)
    S = jnp.dot(L, ksum, preferred_element_type=_F32)
    m = jnp.max(S, axis=1, keepdims=True)
    e = jnp.exp(S - m)
    out_ref[...] = (S - m) - jnp.log(jnp.sum(e, axis=1, keepdims=True))


def kernel(support, query, conv_w1, conv_b1, conv_w2, conv_b2, conv_w3,
           conv_b3, conv_w4, conv_b4, bn_g1, bn_be1, bn_g2, bn_be2, bn_g3,
           bn_be3, bn_g4, bn_be4, att_w, att_b, gru_wi0, gru_wh0, gru_bi0,
           gru_bh0, gru_wi1, gru_wh1, gru_bi1, gru_bh1, trans_w, trans_b):
    del trans_b

    imgs = jnp.concatenate(
        [support.reshape(25, 224, 224), query.reshape(15, 224, 224)], axis=0)
    xpad = jnp.pad(imgs, ((0, 0), (1, 1), (1, 1)))
    par = xpad.reshape(40, 113, 2, 113, 2).transpose(0, 2, 4, 1, 3)
    taps = [par[:, dy & 1, dx & 1, dy // 2:dy // 2 + 112, dx // 2:dx // 2 + 112]
            for dy in range(3) for dx in range(3)]
    x9 = jnp.stack(taps, axis=1).reshape(40, 9, 12544)

    def stp(g, be, b):
        s = g * _BN_INV
        return jnp.stack([s, b * s + be])

    w9 = conv_w1.reshape(32, 9).T
    st1 = stp(bn_g1, bn_be1, conv_b1)
    w2r = conv_w2.transpose(2, 3, 1, 0).reshape(9, 32, 64)
    st2 = stp(bn_g2, bn_be2, conv_b2)
    w3r = conv_w3.transpose(2, 3, 1, 0).reshape(9, 64, 128)
    st3 = stp(bn_g3, bn_be3, conv_b3)
    w4r = conv_w4.transpose(2, 3, 1, 0).reshape(9, 128, 256)
    st4 = stp(bn_g4, bn_be4, conv_b4)
    attwT = att_w.T
    attb2 = att_b[None, :]
    wA = gru_wi0[:, :256].T
    wB = gru_wi0[:, 256:].T
    bi0 = gru_bi0[None, :]

    full = lambda shape: pl.BlockSpec(shape, lambda c, i: (0,) * len(shape))
    img = lambda nd: (lambda c, i: (c * 20 + i,) + (0,) * (nd - 1))
    attp, pA, pB = pl.pallas_call(
        _enc_body,
        grid=(2, 20),
        in_specs=[
            pl.BlockSpec((1, 9, 12544), img(3)),
            full((9, 32)), full((2, 32)),
            full((9, 32, 64)), full((2, 64)),
            full((9, 64, 128)), full((2, 128)),
            full((9, 128, 256)), full((2, 256)),
            full((256, 64)), full((1, 64)),
            full((256, 384)), full((256, 384)), full((1, 384)),
        ],
        out_specs=[
            pl.BlockSpec((1, 49, 64), img(3)),
            pl.BlockSpec((1, 49, 385), img(3)),
            pl.BlockSpec((1, 49, 384), img(3)),
        ],
        out_shape=[
            jax.ShapeDtypeStruct((40, 49, 64), _F32),
            jax.ShapeDtypeStruct((40, 49, 385), _F32),
            jax.ShapeDtypeStruct((40, 49, 384), _F32),
        ],
        compiler_params=pltpu.CompilerParams(
            dimension_semantics=("parallel", "arbitrary")),
        name="encoder",
    )(x9, w9, st1, w2r, st2, w3r, st3, w4r, st4, attwT, attb2, wA, wB, bi0)

    gi = pl.pallas_call(
        _att_body,
        grid=(15, 5),
        in_specs=[
            pl.BlockSpec((1, 49, 64), lambda q, sb: (25 + q, 0, 0)),
            pl.BlockSpec((5, 49, 64), lambda q, sb: (sb, 0, 0)),
            pl.BlockSpec((5, 49, 385), lambda q, sb: (sb, 0, 0)),
            pl.BlockSpec((1, 49, 384), lambda q, sb: (25 + q, 0, 0)),
        ],
        out_specs=pl.BlockSpec((49, 5, 1, 384),
                               lambda q, sb: (0, q * 5 + sb, 0, 0)),
        out_shape=jax.ShapeDtypeStruct((49, 375, 1, 384), _F32),
        compiler_params=pltpu.CompilerParams(
            dimension_semantics=("parallel", "arbitrary")),
        name="attention",
    )(attp, attp, pA, pB)

    gi_t = gi.reshape(49, 375, 384)

    wh0T = gru_wh0.T
    wi1T = gru_wi1.T
    wh1T = gru_wh1.T
    bh0 = gru_bh0[None, :]
    bi1 = gru_bi1[None, :]
    bh1 = gru_bh1[None, :]
    tw = trans_w

    wspec = lambda shape: pl.BlockSpec(shape, lambda t: (0,) * len(shape))
    h_out = pl.pallas_call(
        _gru_body,
        grid=(7,),
        in_specs=[
            pl.BlockSpec((7, 375, 384), lambda t: (t, 0, 0)),
            wspec((128, 384)), wspec((1, 384)),
            wspec((128, 384)), wspec((1, 384)),
            wspec((128, 384)), wspec((1, 384)),
            wspec((1, 128)),
        ],
        out_specs=pl.BlockSpec((1, 375), lambda t: (0, 0)),
        out_shape=jax.ShapeDtypeStruct((1, 375), _F32),
        scratch_shapes=[pltpu.VMEM((375, 128), _F32),
                        pltpu.VMEM((375, 128), _F32)],
        compiler_params=pltpu.CompilerParams(
            dimension_semantics=("arbitrary",)),
        name="gru",
    )(gi_t, wh0T, bh0, wi1T, bi1, wh1T, bh1, tw)

    Lq = h_out.reshape(15, 25)
    return pl.pallas_call(
        _head_body,
        out_shape=jax.ShapeDtypeStruct((15, 5), _F32),
        name="head",
    )(Lq)
